# Initial kernel scaffold; baseline (speedup 1.0000x reference)
import jax
import jax.numpy as jnp
from jax.experimental import pallas as pl


def kernel(target, positive, negatives, target_table, context_table):
    raise NotImplementedError("write your pallas kernel here")


# speedup vs baseline: 4.3129x; 4.3129x over previous
import jax
import jax.numpy as jnp
from jax import lax
from jax.experimental import pallas as pl
from jax.experimental.pallas import tpu as pltpu
from jax.experimental.pallas import tpu_sc as plsc

B = 16384
D = 64
NNEG = 20
L = 16
NC = 2
NS = 16
NW = NC * NS
WB = B // NW
GB = 32
NG = WB // GB
NROW_NEG = GB * NNEG
NCHUNK = NROW_NEG // 128
COLS = 1 + NNEG


def _sc_body(tgt_idx, pos_idx, neg_idx, ttab, ctab, out,
             tix, pix, nix, tbuf, pbuf, nbuf, sbuf, sem):
    wid = lax.axis_index("s") * NC + lax.axis_index("c")
    pltpu.sync_copy(tgt_idx.at[wid], tix)
    pltpu.sync_copy(pos_idx.at[wid], pix)
    pltpu.sync_copy(neg_idx.at[wid], nix)
    lane = lax.iota(jnp.int32, L)
    zero = jnp.zeros((L,), jnp.float32)

    def group(g, carry):
        cps = [
            pltpu.async_copy(ttab.at[tix.at[g]], tbuf, sem),
            pltpu.async_copy(ctab.at[pix.at[g]], pbuf, sem),
        ]
        for j in range(NCHUNK):
            cps.append(pltpu.async_copy(
                ctab.at[nix.at[g, j]], nbuf.at[pl.ds(j * 128, 128)], sem))
        for cp in cps:
            cp.wait()

        for i in range(GB, COLS * GB, L):
            sbuf[pl.ds(i, L)] = zero

        for ls in range(GB // L):
            lb = lane + ls * L
            lb_neg = lb * NNEG
            pos_acc = zero
            for dblk in range(D // L):
                cols = [jnp.full((L,), dblk * L + dd, jnp.int32)
                        for dd in range(L)]
                tv = [plsc.load_gather(tbuf, [lb, cols[dd]])
                      for dd in range(L)]
                for dd in range(L):
                    pos_acc = pos_acc + tv[dd] * plsc.load_gather(
                        pbuf, [lb, cols[dd]])

                def nbody(n, c, _tv=tv, _cols=cols, _lb_neg=lb_neg, _ls=ls):
                    rows = _lb_neg + n
                    acc = _tv[0] * plsc.load_gather(nbuf, [rows, _cols[0]])
                    for dd in range(1, L):
                        acc = acc + _tv[dd] * plsc.load_gather(
                            nbuf, [rows, _cols[dd]])
                    plsc.addupdate(
                        sbuf.at[pl.ds((1 + n) * GB + _ls * L, L)], acc)
                    return c

                lax.fori_loop(0, NNEG, nbody, 0)
            sbuf[pl.ds(ls * L, L)] = -pos_acc

        pltpu.sync_copy(sbuf, out.at[wid, g])
        return carry

    lax.fori_loop(0, NG, group, 0)


def _sc_scores(tgt_idx, pos_idx, neg_idx, ttab, ctab):
    mesh = plsc.VectorSubcoreMesh(core_axis_name="c", subcore_axis_name="s")
    kern = pl.kernel(
        _sc_body,
        out_type=jax.ShapeDtypeStruct((NW, NG, COLS * GB), jnp.float32),
        mesh=mesh,
        compiler_params=pltpu.CompilerParams(
            needs_layout_passes=False, use_tc_tiling_on_sc=False),
        scratch_types=[
            pltpu.VMEM((NG, GB), jnp.int32),
            pltpu.VMEM((NG, GB), jnp.int32),
            pltpu.VMEM((NG, NCHUNK, 128), jnp.int32),
            pltpu.VMEM((GB, D), jnp.float32),
            pltpu.VMEM((GB, D), jnp.float32),
            pltpu.VMEM((NROW_NEG, D), jnp.float32),
            pltpu.VMEM((COLS * GB,), jnp.float32),
            pltpu.SemaphoreType.DMA,
        ],
    )
    return kern(tgt_idx, pos_idx, neg_idx, ttab, ctab)


def _tc_loss_body(s_ref, o_ref):
    x = s_ref[...]
    sp = jnp.maximum(x, 0.0) + jnp.log1p(jnp.exp(-jnp.abs(x)))
    o_ref[0, 0] = jnp.sum(sp) * (1.0 / B)


def _tc_loss(scores2d):
    return pl.pallas_call(
        _tc_loss_body,
        out_shape=jax.ShapeDtypeStruct((1, 1), jnp.float32),
        out_specs=pl.BlockSpec(memory_space=pltpu.SMEM),
    )(scores2d)


def kernel(target, positive, negatives, target_table, context_table):
    tgt = target.astype(jnp.int32).reshape(NW, NG, GB)
    pos = positive.astype(jnp.int32).reshape(NW, NG, GB)
    neg = negatives.astype(jnp.int32).reshape(NW, NG, NCHUNK, 128)
    scores = _sc_scores(tgt, pos, neg, target_table, context_table)
    loss = _tc_loss(scores.reshape(B * COLS // 128, 128))
    return loss[0, 0]


# speedup vs baseline: 5.1266x; 1.1887x over previous
import jax
import jax.numpy as jnp
from jax import lax
from jax.experimental import pallas as pl
from jax.experimental.pallas import tpu as pltpu
from jax.experimental.pallas import tpu_sc as plsc

B = 16384
D = 64
NNEG = 20
L = 16
NC = 2
NS = 16
NW = NC * NS
WB = B // NW
GB = 32
NG = WB // GB
NROW_NEG = GB * NNEG
NCHUNK = NROW_NEG // 128
COLS = 1 + NNEG


def _sc_body(tgt_idx, pos_idx, neg_idx, ttab, ctab, out,
             tix, pix, nix, tbuf, pbuf, nbuf, sbuf, sem):
    wid = lax.axis_index("s") * NC + lax.axis_index("c")
    pltpu.sync_copy(tgt_idx.at[wid], tix)
    pltpu.sync_copy(pos_idx.at[wid], pix)
    pltpu.sync_copy(neg_idx.at[wid], nix)
    lane = lax.iota(jnp.int32, L)
    zero = jnp.zeros((L,), jnp.float32)

    def group(g, carry):
        cps = [
            pltpu.async_copy(ttab.at[tix.at[g]], tbuf, sem),
            pltpu.async_copy(ctab.at[pix.at[g]], pbuf, sem),
        ]
        for j in range(NCHUNK):
            cps.append(pltpu.async_copy(
                ctab.at[nix.at[g, j]], nbuf.at[pl.ds(j * 128, 128)], sem))
        for cp in cps:
            cp.wait()

        for i in range(GB, COLS * GB, L):
            sbuf[pl.ds(i, L)] = zero

        for ls in range(GB // L):
            lb = lane + ls * L
            lb_neg = lb * NNEG
            pos_acc = zero
            for dblk in range(D // L):
                cols = [dblk * L + ((lane + dd) & (L - 1))
                        for dd in range(L)]
                tv = [plsc.load_gather(tbuf, [lb, cols[dd]])
                      for dd in range(L)]
                for dd in range(L):
                    pos_acc = pos_acc + tv[dd] * plsc.load_gather(
                        pbuf, [lb, cols[dd]])

                def nbody(n, c, _tv=tv, _cols=cols, _lb_neg=lb_neg, _ls=ls):
                    rows = _lb_neg + n
                    acc = _tv[0] * plsc.load_gather(nbuf, [rows, _cols[0]])
                    for dd in range(1, L):
                        acc = acc + _tv[dd] * plsc.load_gather(
                            nbuf, [rows, _cols[dd]])
                    plsc.addupdate(
                        sbuf.at[pl.ds((1 + n) * GB + _ls * L, L)], acc)
                    return c

                lax.fori_loop(0, NNEG, nbody, 0)
            sbuf[pl.ds(ls * L, L)] = -pos_acc

        pltpu.sync_copy(sbuf, out.at[wid, g])
        return carry

    lax.fori_loop(0, NG, group, 0)


def _sc_scores(tgt_idx, pos_idx, neg_idx, ttab, ctab):
    mesh = plsc.VectorSubcoreMesh(core_axis_name="c", subcore_axis_name="s")
    kern = pl.kernel(
        _sc_body,
        out_type=jax.ShapeDtypeStruct((NW, NG, COLS * GB), jnp.float32),
        mesh=mesh,
        compiler_params=pltpu.CompilerParams(
            needs_layout_passes=False, use_tc_tiling_on_sc=False),
        scratch_types=[
            pltpu.VMEM((NG, GB), jnp.int32),
            pltpu.VMEM((NG, GB), jnp.int32),
            pltpu.VMEM((NG, NCHUNK, 128), jnp.int32),
            pltpu.VMEM((GB, D), jnp.float32),
            pltpu.VMEM((GB, D), jnp.float32),
            pltpu.VMEM((NROW_NEG, D), jnp.float32),
            pltpu.VMEM((COLS * GB,), jnp.float32),
            pltpu.SemaphoreType.DMA,
        ],
    )
    return kern(tgt_idx, pos_idx, neg_idx, ttab, ctab)


def _tc_loss_body(s_ref, o_ref):
    x = s_ref[...]
    sp = jnp.maximum(x, 0.0) + jnp.log1p(jnp.exp(-jnp.abs(x)))
    o_ref[0, 0] = jnp.sum(sp) * (1.0 / B)


def _tc_loss(scores2d):
    return pl.pallas_call(
        _tc_loss_body,
        out_shape=jax.ShapeDtypeStruct((1, 1), jnp.float32),
        out_specs=pl.BlockSpec(memory_space=pltpu.SMEM),
    )(scores2d)


def kernel(target, positive, negatives, target_table, context_table):
    tgt = target.astype(jnp.int32).reshape(NW, NG, GB)
    pos = positive.astype(jnp.int32).reshape(NW, NG, GB)
    neg = negatives.astype(jnp.int32).reshape(NW, NG, NCHUNK, 128)
    scores = _sc_scores(tgt, pos, neg, target_table, context_table)
    loss = _tc_loss(scores.reshape(B * COLS // 128, 128))
    return loss[0, 0]


# speedup vs baseline: 9.7704x; 1.9058x over previous
import jax
import jax.numpy as jnp
from jax import lax
from jax.experimental import pallas as pl
from jax.experimental.pallas import tpu as pltpu
from jax.experimental.pallas import tpu_sc as plsc

B = 16384
D = 64
V = 1000000
NNEG = 20
L = 16
NC = 2
NS = 16
NW = NC * NS
WB = B // NW
GB = 32
NG = WB // GB
NROW_NEG = GB * NNEG
NCHUNK = NROW_NEG // 128
COLS = 1 + NNEG

NFULL = V // 128
VP = (NFULL + 1) * 128


def _tc_tr_body(x_ref, o_ref):
    x = x_ref[...]
    eye = jnp.eye(D, dtype=jnp.float32)
    xt = lax.dot_general(x, eye, (((0,), (0,)), ((), ())),
                         preferred_element_type=jnp.float32)
    o_ref[...] = jnp.concatenate([xt, jnp.zeros_like(xt)], axis=1)


def _tc_transpose(x):
    swt = 4096
    return pl.pallas_call(
        _tc_tr_body,
        grid=(VP // swt + 1,),
        in_specs=[pl.BlockSpec((D, swt), lambda j: (0, j))],
        out_specs=pl.BlockSpec((swt, 2 * D), lambda j: (j, 0)),
        out_shape=jax.ShapeDtypeStruct((VP, 2 * D), jnp.float32),
    )(x)


def _tr_body(ca, cout, ib0, ib1, ob0, ob1,
             si0, si1, so0, so1):
    wid = lax.axis_index("s") * NC + lax.axis_index("c")
    lane = lax.iota(jnp.int32, L)
    perm = [(lane + j) & (L - 1) for j in range(L)]
    SW = 256
    nvt = -(-VP // SW)
    ntask = nvt

    def stripe(tid):
        return pl.multiple_of(jnp.minimum(tid * SW, VP - SW), 128)

    def in_copies(src, tid, ib, sem):
        vm = stripe(tid)
        return [pltpu.make_async_copy(
            src.at[pl.ds(dt * 8, 8), pl.ds(vm, SW)],
            ib.at[pl.ds(dt * 8, 8), :], sem) for dt in range(D // 8)]

    def fire_in(tid, ib, sem):
        @pl.when(tid < ntask)
        def _():
            for cp in in_copies(ca, tid, ib, sem):
                cp.start()

    def wait_in(tid, ib, sem):
        @pl.when(tid < ntask)
        def _():
            for cp in in_copies(ca, tid, ib, sem):
                cp.wait()

    def out_copy(dst, tid, ob, sem):
        vm = stripe(tid)
        return pltpu.make_async_copy(ob, dst.at[pl.ds(vm * D, SW * D)], sem)

    def fire_out(tid, ob, sem):
        @pl.when(tid < ntask)
        def _():
            out_copy(cout, tid, ob, sem).start()

    def wait_out(tid, ob, sem):
        @pl.when((tid >= 0) & (tid < ntask))
        def _():
            out_copy(cout, tid, ob, sem).wait()

    def transpose_block(ib, ob):
        def tb(i, c):
            vb = i >> 2
            dg = i & 3
            sv = vb * L + lane
            svd = sv * D
            for j in range(L):
                dv = dg * L + perm[j]
                val = plsc.load_gather(ib, [dv, sv])
                plsc.store_scatter(ob, [svd + dv], val)
            return c

        lax.fori_loop(0, (SW // L) * (D // L), tb, 0)

    def slot(tid, ib, ob, sin, sin_next, ib_next, sout):
        fire_in(tid + NW, ib_next, sin_next)
        wait_in(tid, ib, sin)
        wait_out(tid - 2 * NW, ob, sout)
        transpose_block(ib, ob)
        fire_out(tid, ob, sout)

    fire_in(wid, ib0, si0)

    def pipe(m, c):
        t0 = wid + (2 * m) * NW
        slot(t0, ib0, ob0, si0, si1, ib1, so0)
        slot(t0 + NW, ib1, ob1, si1, si0, ib0, so1)
        return c

    nslots = -(-ntask // NW)
    lax.fori_loop(0, (nslots + 1) // 2, pipe, 0)
    last = wid + (2 * ((nslots + 1) // 2) - 2) * NW
    wait_out(last, ob0, so0)
    wait_out(last + NW, ob1, so1)


def _transpose_ctx(ctab_t):
    mesh = plsc.VectorSubcoreMesh(core_axis_name="c", subcore_axis_name="s")
    kern = pl.kernel(
        _tr_body,
        out_type=jax.ShapeDtypeStruct((VP * D,), jnp.float32),
        mesh=mesh,
        compiler_params=pltpu.CompilerParams(
            needs_layout_passes=False, use_tc_tiling_on_sc=True),
        scratch_types=[
            pltpu.VMEM((D, 256), jnp.float32),
            pltpu.VMEM((D, 256), jnp.float32),
            pltpu.VMEM((256 * D,), jnp.float32),
            pltpu.VMEM((256 * D,), jnp.float32),
            pltpu.SemaphoreType.DMA,
            pltpu.SemaphoreType.DMA,
            pltpu.SemaphoreType.DMA,
            pltpu.SemaphoreType.DMA,
        ],
    )
    return kern(ctab_t)


def _sc_body(tgt_idx, pos_idx, neg_idx, ttab, ctab, out,
             tix, pix, nix, tbuf, pbuf, nbuf, sbuf, sem):
    wid = lax.axis_index("s") * NC + lax.axis_index("c")
    pltpu.sync_copy(tgt_idx.at[wid], tix)
    pltpu.sync_copy(pos_idx.at[wid], pix)
    pltpu.sync_copy(neg_idx.at[wid], nix)
    lane = lax.iota(jnp.int32, L)
    zero = jnp.zeros((L,), jnp.float32)

    def group(g, carry):
        cps = [
            pltpu.async_copy(ttab.at[tix.at[g]], tbuf, sem),
            pltpu.async_copy(ctab.at[pix.at[g]], pbuf, sem),
        ]
        for j in range(NCHUNK):
            cps.append(pltpu.async_copy(
                ctab.at[nix.at[g, j]], nbuf.at[pl.ds(j * 128, 128)], sem))
        for cp in cps:
            cp.wait()

        def zbody(i, c):
            sbuf[pl.ds(i * L, L)] = zero
            return c

        lax.fori_loop(GB // L, COLS * GB // L, zbody, 0)

        def lsbody(ls, c):
            lb = lane + ls * L
            lb_neg = lb * NNEG

            def dbody(dblk, pos_acc):
                cols = [dblk * L + ((lane + dd) & (L - 1))
                        for dd in range(L)]
                tv = [plsc.load_gather(tbuf, [lb, cols[dd]])
                      for dd in range(L)]
                for dd in range(L):
                    pos_acc = pos_acc + tv[dd] * plsc.load_gather(
                        pbuf, [lb, cols[dd]])

                def nbody(n, c2, _tv=tv, _cols=cols):
                    rows = lb_neg + n
                    acc = _tv[0] * plsc.load_gather(nbuf, [rows, _cols[0]])
                    for dd in range(1, L):
                        acc = acc + _tv[dd] * plsc.load_gather(
                            nbuf, [rows, _cols[dd]])
                    plsc.addupdate(
                        sbuf.at[pl.ds((1 + n) * GB + ls * L, L)], acc)
                    return c2

                lax.fori_loop(0, NNEG, nbody, 0)
                return pos_acc

            pos_acc = lax.fori_loop(0, D // L, dbody, zero)
            sbuf[pl.ds(ls * L, L)] = -pos_acc
            return c

        lax.fori_loop(0, GB // L, lsbody, 0)

        pltpu.sync_copy(sbuf, out.at[wid, g])
        return carry

    lax.fori_loop(0, NG, group, 0)


def _sc_scores(tgt_idx, pos_idx, neg_idx, ttab, ctab):
    mesh = plsc.VectorSubcoreMesh(core_axis_name="c", subcore_axis_name="s")
    kern = pl.kernel(
        _sc_body,
        out_type=jax.ShapeDtypeStruct((NW, NG, COLS * GB), jnp.float32),
        mesh=mesh,
        compiler_params=pltpu.CompilerParams(
            needs_layout_passes=False, use_tc_tiling_on_sc=False),
        scratch_types=[
            pltpu.VMEM((NG, GB), jnp.int32),
            pltpu.VMEM((NG, GB), jnp.int32),
            pltpu.VMEM((NG, NCHUNK, 128), jnp.int32),
            pltpu.VMEM((GB, 2 * D), jnp.float32),
            pltpu.VMEM((GB, D), jnp.float32),
            pltpu.VMEM((NROW_NEG, D), jnp.float32),
            pltpu.VMEM((COLS * GB,), jnp.float32),
            pltpu.SemaphoreType.DMA,
        ],
    )
    return kern(tgt_idx, pos_idx, neg_idx, ttab, ctab)


def _tc_loss_body(s_ref, o_ref):
    x = s_ref[...]
    sp = jnp.maximum(x, 0.0) + jnp.log1p(jnp.exp(-jnp.abs(x)))
    o_ref[0, 0] = jnp.sum(sp) * (1.0 / B)


def _tc_loss(scores2d):
    return pl.pallas_call(
        _tc_loss_body,
        out_shape=jax.ShapeDtypeStruct((1, 1), jnp.float32),
        out_specs=pl.BlockSpec(memory_space=pltpu.SMEM),
    )(scores2d)


def kernel(target, positive, negatives, target_table, context_table):
    tgt = target.astype(jnp.int32).reshape(NW, NG, GB)
    pos = positive.astype(jnp.int32).reshape(NW, NG, GB)
    neg = negatives.astype(jnp.int32).reshape(NW, NG, NCHUNK, 128)
    tlin = _tc_transpose(target_table.T)
    clin = _transpose_ctx(context_table.T)
    scores = _sc_scores(tgt, pos, neg,
                        tlin.reshape(-1).reshape(VP, 2 * D),
                        clin.reshape(VP, D))
    loss = _tc_loss(scores.reshape(B * COLS // 128, 128))
    return loss[0, 0]
